# baseline (speedup 1.0000x reference)
import dataclasses

import jax
import jax.numpy as jnp
from jax import lax
from jax.experimental import pallas as pl
from jax.experimental.pallas import tpu as pltpu
from jax.experimental.pallas import tpu_sc as plsc

VOCAB = 1000000
EMBED = 64
DPAD = 128
B = 4096
L = 50

NC = 2
NS = 16
NW = NC * NS
BW = B // NW
RW = BW * L
CB = 8
CHUNK = CB * L
NCHUNK = BW // CB
LPAD = 64
NG = L // 16


def _sc_kernel(crow_hbm, coff_hbm, urow_hbm, uoff_hbm, ev_hbm, eu_hbm,
               out_hbm, crow_v, coff_v, v_rows, urow_v, uoff_v, u_rows,
               s_tile, o_all, sem):
    wid = lax.axis_index("s") * NC + lax.axis_index("c")
    iota = lax.iota(jnp.int32, 16)

    pltpu.sync_copy(crow_hbm.at[pl.ds(wid * BW, BW)], crow_v)
    pltpu.sync_copy(coff_hbm.at[pl.ds(wid * BW, BW)], coff_v.at[pl.ds(0, BW)])
    pltpu.sync_copy(urow_hbm.at[pl.ds(wid * RW, RW)], urow_v)
    pltpu.sync_copy(uoff_hbm.at[pl.ds(wid * RW, RW)], uoff_v)

    pltpu.async_copy(ev_hbm.at[crow_v], v_rows, sem).wait()

    def dot_row(r, off, v0, v1, v2, v3):
        acc = u_rows[r, pl.ds(off, 16)] * v0
        acc = acc + u_rows[r, pl.ds(off + 16, 16)] * v1
        acc = acc + u_rows[r, pl.ds(off + 32, 16)] * v2
        acc = acc + u_rows[r, pl.ds(off + 48, 16)] * v3
        return acc

    def reduce_tile():
        out16 = s_tile[pl.ds(0, 16)]
        for k in range(1, 16):
            out16 = out16 + s_tile[pl.ds(16 * k, 16)]
        return out16

    @pl.loop(0, NCHUNK)
    def _(c):
        pltpu.async_copy(
            eu_hbm.at[urow_v.at[pl.ds(c * CHUNK, CHUNK)]], u_rows, sem
        ).wait()

        for b in range(CB):
            bb = c * CB + b
            co = coff_v[pl.ds(bb, 16)][0]
            v0 = v_rows[bb, pl.ds(co, 16)]
            v1 = v_rows[bb, pl.ds(co + 16, 16)]
            v2 = v_rows[bb, pl.ds(co + 32, 16)]
            v3 = v_rows[bb, pl.ds(co + 48, 16)]
            for g in range(NG):
                offs = uoff_v[pl.ds(c * CHUNK + b * L + 16 * g, 16)]
                for j in range(16):
                    r = b * L + 16 * g + j
                    acc = dot_row(r, offs[j], v0, v1, v2, v3)
                    plsc.store_scatter(s_tile, [iota * 16 + j], acc)
                o16 = reduce_tile()
                o_all[pl.ds(bb * LPAD + 16 * g, 16)] = o16

        tail_r = (iota // 2) * L + 48 + (iota % 2)
        toffs = plsc.load_gather(uoff_v, [c * CHUNK + tail_r])
        for j in range(16):
            b = j // 2
            if j % 2 == 0:
                tco = coff_v[pl.ds(c * CB + b, 16)][0]
                tv0 = v_rows[c * CB + b, pl.ds(tco, 16)]
                tv1 = v_rows[c * CB + b, pl.ds(tco + 16, 16)]
                tv2 = v_rows[c * CB + b, pl.ds(tco + 32, 16)]
                tv3 = v_rows[c * CB + b, pl.ds(tco + 48, 16)]
            r = b * L + 48 + (j % 2)
            acc = dot_row(r, toffs[j], tv0, tv1, tv2, tv3)
            plsc.store_scatter(s_tile, [iota * 16 + j], acc)
        o16 = reduce_tile()
        dest = (c * CB + (iota // 2)) * LPAD + 48 + (iota % 2)
        plsc.store_scatter(o_all, [dest], o16)

    pltpu.sync_copy(o_all, out_hbm.at[pl.ds(wid * BW * LPAD, BW * LPAD)])


def kernel(center, context_negative, embed_v, embed_u):
    cflat = center.reshape(B)
    uflat = context_negative.reshape(B * L)
    crow, coff = cflat >> 1, (cflat & 1) * EMBED
    urow, uoff = uflat >> 1, (uflat & 1) * EMBED
    ev2 = embed_v.reshape(VOCAB // 2, DPAD)
    eu2 = embed_u.reshape(VOCAB // 2, DPAD)

    mesh = plsc.VectorSubcoreMesh(core_axis_name="c", subcore_axis_name="s")
    cp = pltpu.CompilerParams()
    if "needs_layout_passes" in pltpu.CompilerParams.__dataclass_fields__:
        cp = dataclasses.replace(cp, needs_layout_passes=False)
    k = pl.kernel(
        _sc_kernel,
        compiler_params=cp,
        out_type=jax.ShapeDtypeStruct((B * LPAD,), jnp.float32),
        mesh=mesh,
        scratch_types=[
            pltpu.VMEM((BW,), jnp.int32),
            pltpu.VMEM((BW + 16,), jnp.int32),
            pltpu.VMEM((BW, DPAD), jnp.float32),
            pltpu.VMEM((RW,), jnp.int32),
            pltpu.VMEM((RW,), jnp.int32),
            pltpu.VMEM((CHUNK, DPAD), jnp.float32),
            pltpu.VMEM((256,), jnp.float32),
            pltpu.VMEM((BW * LPAD,), jnp.float32),
            pltpu.SemaphoreType.DMA,
        ],
    )
    out = k(crow, coff, urow, uoff, ev2, eu2)
    return out.reshape(B, LPAD)[:, :L].reshape(B, 1, L)


# speedup vs baseline: 1.0225x; 1.0225x over previous
import dataclasses

import jax
import jax.numpy as jnp
from jax import lax
from jax.experimental import pallas as pl
from jax.experimental.pallas import tpu as pltpu
from jax.experimental.pallas import tpu_sc as plsc

VOCAB = 1000000
EMBED = 64
B = 4096
L = 50

NC = 2
NS = 16
NW = NC * NS
BW = B // NW
RW = BW * L
CB = 8
CHUNK = CB * L
NCHUNK = BW // CB
LPAD = 64
NG = L // 16


def _sc_kernel(crow_hbm, urow_hbm, ev_hbm, eu_hbm,
               out_hbm, crow_v, v_rows, urow_v, u_rows,
               s_tile, o_all, sem):
    wid = lax.axis_index("s") * NC + lax.axis_index("c")
    iota = lax.iota(jnp.int32, 16)

    pltpu.sync_copy(crow_hbm.at[pl.ds(wid * BW, BW)], crow_v)
    pltpu.sync_copy(urow_hbm.at[pl.ds(wid * RW, RW)], urow_v)

    pltpu.async_copy(ev_hbm.at[crow_v], v_rows, sem).wait()

    def dot_row(r, v0, v1, v2, v3):
        acc = u_rows[r, pl.ds(0, 16)] * v0
        acc = acc + u_rows[r, pl.ds(16, 16)] * v1
        acc = acc + u_rows[r, pl.ds(32, 16)] * v2
        acc = acc + u_rows[r, pl.ds(48, 16)] * v3
        return acc

    def reduce_tile():
        out16 = s_tile[pl.ds(0, 16)]
        for k in range(1, 16):
            out16 = out16 + s_tile[pl.ds(16 * k, 16)]
        return out16

    @pl.loop(0, NCHUNK)
    def _(c):
        pltpu.async_copy(
            eu_hbm.at[urow_v.at[pl.ds(c * CHUNK, CHUNK)]], u_rows, sem
        ).wait()

        for b in range(CB):
            bb = c * CB + b
            v0 = v_rows[bb, pl.ds(0, 16)]
            v1 = v_rows[bb, pl.ds(16, 16)]
            v2 = v_rows[bb, pl.ds(32, 16)]
            v3 = v_rows[bb, pl.ds(48, 16)]
            for g in range(NG):
                for j in range(16):
                    r = b * L + 16 * g + j
                    acc = dot_row(r, v0, v1, v2, v3)
                    plsc.store_scatter(s_tile, [iota * 16 + j], acc)
                o16 = reduce_tile()
                o_all[pl.ds(bb * LPAD + 16 * g, 16)] = o16

        for j in range(16):
            b = j // 2
            if j % 2 == 0:
                tv0 = v_rows[c * CB + b, pl.ds(0, 16)]
                tv1 = v_rows[c * CB + b, pl.ds(16, 16)]
                tv2 = v_rows[c * CB + b, pl.ds(32, 16)]
                tv3 = v_rows[c * CB + b, pl.ds(48, 16)]
            r = b * L + 48 + (j % 2)
            acc = dot_row(r, tv0, tv1, tv2, tv3)
            plsc.store_scatter(s_tile, [iota * 16 + j], acc)
        o16 = reduce_tile()
        dest = (c * CB + (iota // 2)) * LPAD + 48 + (iota % 2)
        plsc.store_scatter(o_all, [dest], o16)

    pltpu.sync_copy(o_all, out_hbm.at[pl.ds(wid * BW * LPAD, BW * LPAD)])


def kernel(center, context_negative, embed_v, embed_u):
    crow = center.reshape(B)
    urow = context_negative.reshape(B * L)

    mesh = plsc.VectorSubcoreMesh(core_axis_name="c", subcore_axis_name="s")
    cp = pltpu.CompilerParams()
    fields = pltpu.CompilerParams.__dataclass_fields__
    if "needs_layout_passes" in fields:
        cp = dataclasses.replace(cp, needs_layout_passes=False)
    if "use_tc_tiling_on_sc" in fields:
        cp = dataclasses.replace(cp, use_tc_tiling_on_sc=False)
    k = pl.kernel(
        _sc_kernel,
        compiler_params=cp,
        out_type=jax.ShapeDtypeStruct((B * LPAD,), jnp.float32),
        mesh=mesh,
        scratch_types=[
            pltpu.VMEM((BW,), jnp.int32),
            pltpu.VMEM((BW, EMBED), jnp.float32),
            pltpu.VMEM((RW,), jnp.int32),
            pltpu.VMEM((CHUNK, EMBED), jnp.float32),
            pltpu.VMEM((256,), jnp.float32),
            pltpu.VMEM((BW * LPAD,), jnp.float32),
            pltpu.SemaphoreType.DMA,
        ],
    )
    out = k(crow, urow, embed_v, embed_u)
    return out.reshape(B, LPAD)[:, :L].reshape(B, 1, L)


# speedup vs baseline: 1.0403x; 1.0174x over previous
import dataclasses

import jax
import jax.numpy as jnp
from jax import lax
from jax.experimental import pallas as pl
from jax.experimental.pallas import tpu as pltpu
from jax.experimental.pallas import tpu_sc as plsc

VOCAB = 1000000
EMBED = 64
B = 4096
L = 50

NC = 2
NS = 16
NW = NC * NS
BW = B // NW
RW = BW * L
CB = 4
CHUNK = CB * L
NCHUNK = BW // CB
NPAIR = NCHUNK // 2
LPAD = 64
NG = L // 16
SSTRIDE = 17
TRASH = BW * LPAD


def _sc_kernel(crow_hbm, urow_hbm, ev_hbm, eu_hbm,
               out_hbm, crow_v, v_rows, urow_v, u0, u1,
               s_tile, o_all, sem_v, sem0, sem1):
    wid = lax.axis_index("s") * NC + lax.axis_index("c")
    iota = lax.iota(jnp.int32, 16)

    pltpu.sync_copy(crow_hbm.at[pl.ds(wid * BW, BW)], crow_v)
    pltpu.sync_copy(urow_hbm.at[pl.ds(wid * RW, RW)], urow_v)

    pltpu.async_copy(ev_hbm.at[crow_v], v_rows, sem_v).wait()

    def dot_row(u_rows, r, v0, v1, v2, v3):
        acc = u_rows[r, pl.ds(0, 16)] * v0
        acc = acc + u_rows[r, pl.ds(16, 16)] * v1
        acc = acc + u_rows[r, pl.ds(32, 16)] * v2
        acc = acc + u_rows[r, pl.ds(48, 16)] * v3
        return acc

    def reduce_tile():
        out16 = s_tile[pl.ds(0, 16)]
        for k in range(1, 16):
            out16 = out16 + s_tile[pl.ds(SSTRIDE * k, 16)]
        return out16

    def compute_chunk(u_rows, c):
        for b in range(CB):
            bb = c * CB + b
            v0 = v_rows[bb, pl.ds(0, 16)]
            v1 = v_rows[bb, pl.ds(16, 16)]
            v2 = v_rows[bb, pl.ds(32, 16)]
            v3 = v_rows[bb, pl.ds(48, 16)]
            for g in range(NG):
                for j in range(16):
                    r = b * L + 16 * g + j
                    acc = dot_row(u_rows, r, v0, v1, v2, v3)
                    plsc.store_scatter(s_tile, [iota * SSTRIDE + j], acc)
                o16 = reduce_tile()
                o_all[pl.ds(bb * LPAD + 16 * g, 16)] = o16

        for j in range(8):
            b = j // 2
            if j % 2 == 0:
                tv0 = v_rows[c * CB + b, pl.ds(0, 16)]
                tv1 = v_rows[c * CB + b, pl.ds(16, 16)]
                tv2 = v_rows[c * CB + b, pl.ds(32, 16)]
                tv3 = v_rows[c * CB + b, pl.ds(48, 16)]
            r = b * L + 48 + (j % 2)
            acc = dot_row(u_rows, r, tv0, tv1, tv2, tv3)
            plsc.store_scatter(s_tile, [iota * SSTRIDE + j], acc)
        o16 = reduce_tile()
        dest = jnp.where(
            iota < 8,
            (c * CB + iota // 2) * LPAD + 48 + (iota % 2),
            TRASH + iota,
        )
        plsc.store_scatter(o_all, [dest], o16)

    def gather_chunk(c, buf, sem):
        return pltpu.async_copy(
            eu_hbm.at[urow_v.at[pl.ds(c * CHUNK, CHUNK)]], buf, sem
        )

    gather_chunk(0, u0, sem0)

    @pl.loop(0, NPAIR)
    def _(g):
        a = 2 * g
        pltpu.make_async_copy(
            eu_hbm.at[urow_v.at[pl.ds(a * CHUNK, CHUNK)]], u0, sem0
        ).wait()
        gather_chunk(a + 1, u1, sem1)
        compute_chunk(u0, a)
        nxt = jnp.minimum(a + 2, NCHUNK - 1)
        gather_chunk(nxt, u0, sem0)
        pltpu.make_async_copy(
            eu_hbm.at[urow_v.at[pl.ds((a + 1) * CHUNK, CHUNK)]], u1, sem1
        ).wait()
        compute_chunk(u1, a + 1)

    pltpu.make_async_copy(
        eu_hbm.at[urow_v.at[pl.ds((NCHUNK - 1) * CHUNK, CHUNK)]], u0, sem0
    ).wait()

    pltpu.sync_copy(o_all.at[pl.ds(0, BW * LPAD)],
                    out_hbm.at[pl.ds(wid * BW * LPAD, BW * LPAD)])


def kernel(center, context_negative, embed_v, embed_u):
    crow = center.reshape(B)
    urow = context_negative.reshape(B * L)

    mesh = plsc.VectorSubcoreMesh(core_axis_name="c", subcore_axis_name="s")
    cp = pltpu.CompilerParams()
    fields = pltpu.CompilerParams.__dataclass_fields__
    if "needs_layout_passes" in fields:
        cp = dataclasses.replace(cp, needs_layout_passes=False)
    if "use_tc_tiling_on_sc" in fields:
        cp = dataclasses.replace(cp, use_tc_tiling_on_sc=False)
    k = pl.kernel(
        _sc_kernel,
        compiler_params=cp,
        out_type=jax.ShapeDtypeStruct((B * LPAD,), jnp.float32),
        mesh=mesh,
        scratch_types=[
            pltpu.VMEM((BW,), jnp.int32),
            pltpu.VMEM((BW, EMBED), jnp.float32),
            pltpu.VMEM((RW,), jnp.int32),
            pltpu.VMEM((CHUNK, EMBED), jnp.float32),
            pltpu.VMEM((CHUNK, EMBED), jnp.float32),
            pltpu.VMEM((SSTRIDE * 16,), jnp.float32),
            pltpu.VMEM((BW * LPAD + 16,), jnp.float32),
            pltpu.SemaphoreType.DMA,
            pltpu.SemaphoreType.DMA,
            pltpu.SemaphoreType.DMA,
        ],
    )
    out = k(crow, urow, embed_v, embed_u)
    return out.reshape(B, LPAD)[:, :L].reshape(B, 1, L)
